# baseline (speedup 1.0000x reference)
import functools

import jax
import jax.numpy as jnp
from jax import lax
from jax.experimental import pallas as pl
from jax.experimental.pallas import tpu as pltpu
from jax.experimental.pallas import tpu_sc as plsc

NUM_EMBEDDINGS = 100000
EMBED = 64
BATCH = 4096
FIELDS = 100

NC = 2
NS = 16
NW = NC * NS

E_PER_W = EMBED // NW
FH = FIELDS // 2


def _gather_body(off, idx_hbm, wT_hbm, out_hbm, idx_sh, tbl_v, idx_v, out_v,
                 tsem, isem, osem):
    c = lax.axis_index("c")
    s = lax.axis_index("s")
    wid = s * NC + c

    def wait_idx(b):
        pltpu.make_async_copy(
            idx_sh.at[pl.ds(0, BATCH)], idx_v.at[b], isem.at[b]
        ).wait()

    def wait_out(b):
        pltpu.make_async_copy(out_v.at[b], out_hbm.at[0, 0], osem.at[b]).wait()

    @pl.when(s == 0)
    def _():
        pltpu.sync_copy(idx_hbm.at[pl.ds(off * BATCH, FH * BATCH)], idx_sh)

    plsc.subcore_barrier()

    for t in range(E_PER_W):
        e = wid * E_PER_W + t
        pltpu.async_copy(wT_hbm.at[e], tbl_v, tsem)
        pltpu.make_async_copy(wT_hbm.at[0], tbl_v, tsem).wait()

        for b in range(2):
            pltpu.async_copy(
                idx_sh.at[pl.ds(b * BATCH, BATCH)], idx_v.at[b], isem.at[b]
            )

        @pl.loop(0, FH, step=2)
        def _f(f0):
            for b in range(2):
                fl = f0 + b
                iv = idx_v.at[b]
                ov = out_v.at[b]
                wait_idx(b)

                @pl.when(f0 + b >= 2)
                def _():
                    wait_out(b)

                @plsc.parallel_loop(0, BATCH, step=16, unroll=8)
                def _i(i):
                    ov[pl.ds(i, 16)] = plsc.load_gather(tbl_v, [iv[pl.ds(i, 16)]])

                pltpu.async_copy(ov, out_hbm.at[fl, e], osem.at[b])

                @pl.when(f0 + b + 2 < FH)
                def _():
                    pltpu.async_copy(
                        idx_sh.at[pl.ds((fl + 2) * BATCH, BATCH)],
                        idx_v.at[b],
                        isem.at[b],
                    )

        for b in range(2):
            wait_out(b)


def _gather_half(idxT, wT, off):
    mesh = plsc.VectorSubcoreMesh(core_axis_name="c", subcore_axis_name="s")
    fn = pl.kernel(
        functools.partial(_gather_body, off),
        out_type=jax.ShapeDtypeStruct((FH, EMBED, BATCH), jnp.float32),
        mesh=mesh,
        scratch_types=[
            pltpu.VMEM_SHARED((FH * BATCH,), jnp.int32),
            pltpu.VMEM((NUM_EMBEDDINGS,), jnp.float32),
            pltpu.VMEM((2, BATCH), jnp.int32),
            pltpu.VMEM((2, BATCH), jnp.float32),
            pltpu.SemaphoreType.DMA,
            pltpu.SemaphoreType.DMA((2,)),
            pltpu.SemaphoreType.DMA((2,)),
        ],
        compiler_params=pltpu.CompilerParams(
            use_tc_tiling_on_sc=False, needs_layout_passes=False
        ),
    )
    return fn(idxT, wT)


@jax.jit
def _gather_sc(idxT, wT):
    o0 = _gather_half(idxT, wT, 0)
    o1 = _gather_half(idxT, wT, FH)
    return jnp.concatenate([o0, o1], axis=0)


def kernel(x, indices, weight):
    idxT = indices.astype(jnp.int32).T.reshape(-1)
    wT = weight.T
    out = _gather_sc(idxT, wT)
    return out.transpose(2, 0, 1)


# speedup vs baseline: 1.8196x; 1.8196x over previous
import jax
import jax.numpy as jnp
from jax import lax
from jax.experimental import pallas as pl
from jax.experimental.pallas import tpu as pltpu
from jax.experimental.pallas import tpu_sc as plsc
from jax.experimental.layout import Format, Layout, with_layout_constraint

NUM_EMBEDDINGS = 100000
EMBED = 64
BATCH = 4096
FIELDS = 100

NC = 2
NS = 16
NW = NC * NS

E_PER_W = EMBED // NW
FBLOCKS = 2
FB_FIELDS = FIELDS // FBLOCKS


def _gather_body(idx_hbm, wT_hbm, out_hbm, idx_sh, tbl_v, idx_v, out_v, tsem, isem, osem):
    c = lax.axis_index("c")
    s = lax.axis_index("s")
    wid = s * NC + c

    def wait_idx(b):
        pltpu.make_async_copy(
            idx_sh.at[pl.ds(0, BATCH)], idx_v.at[b], isem.at[b]
        ).wait()

    def wait_out(b):
        pltpu.make_async_copy(
            out_v.at[b], out_hbm.at[0, 0, :, 0, :], osem.at[b]
        ).wait()

    for t in range(E_PER_W):
        e = wid * E_PER_W + t
        er = e // 8
        r = e % 8
        pltpu.async_copy(wT_hbm.at[e], tbl_v, tsem)
        pltpu.make_async_copy(wT_hbm.at[0], tbl_v, tsem).wait()

        for fb in range(FBLOCKS):
            f_base = fb * FB_FIELDS
            plsc.subcore_barrier()

            @pl.when(s == 0)
            def _():
                pltpu.sync_copy(
                    idx_hbm.at[pl.ds(f_base * BATCH, FB_FIELDS * BATCH)], idx_sh
                )

            plsc.subcore_barrier()

            for b in range(2):
                pltpu.async_copy(
                    idx_sh.at[pl.ds(b * BATCH, BATCH)], idx_v.at[b], isem.at[b]
                )

            @pl.loop(0, FB_FIELDS, step=2)
            def _f(f0):
                for b in range(2):
                    fl = f0 + b
                    iv = idx_v.at[b]
                    ov = out_v.at[b]
                    wait_idx(b)

                    @pl.when(f0 + b >= 2)
                    def _():
                        wait_out(b)

                    @plsc.parallel_loop(0, 32, unroll=2)
                    def _i(jj):
                        for k in range(8):
                            ov[jj, pl.ds(k * 16, 16)] = plsc.load_gather(
                                tbl_v, [iv[pl.ds(jj * 128 + k * 16, 16)]]
                            )

                    pltpu.async_copy(
                        ov, out_hbm.at[f_base + fl, er, :, r, :], osem.at[b]
                    )

                    @pl.when(f0 + b + 2 < FB_FIELDS)
                    def _():
                        pltpu.async_copy(
                            idx_sh.at[pl.ds((fl + 2) * BATCH, BATCH)],
                            idx_v.at[b],
                            isem.at[b],
                        )

            for b in range(2):
                wait_out(b)


@jax.jit
def _gather_sc(idxT, wT):
    mesh = plsc.VectorSubcoreMesh(core_axis_name="c", subcore_axis_name="s")
    fn = pl.kernel(
        _gather_body,
        out_type=jax.ShapeDtypeStruct((FIELDS, 8, 32, 8, 128), jnp.float32),
        mesh=mesh,
        scratch_types=[
            pltpu.VMEM_SHARED((FB_FIELDS * BATCH,), jnp.int32),
            pltpu.VMEM((NUM_EMBEDDINGS,), jnp.float32),
            pltpu.VMEM((2, BATCH), jnp.int32),
            pltpu.VMEM((2, 32, 128), jnp.float32),
            pltpu.SemaphoreType.DMA,
            pltpu.SemaphoreType.DMA((2,)),
            pltpu.SemaphoreType.DMA((2,)),
        ],
        compiler_params=pltpu.CompilerParams(
            use_tc_tiling_on_sc=False, needs_layout_passes=False
        ),
    )
    return fn(idxT, wT)


def kernel(x, indices, weight):
    idxT = indices.astype(jnp.int32).T.reshape(-1)
    wT = weight.T
    out = _gather_sc(idxT, wT)
    return out.transpose(2, 4, 0, 1, 3).reshape(BATCH, FIELDS, EMBED)


# speedup vs baseline: 1.8581x; 1.0211x over previous
import jax
import jax.numpy as jnp
from jax import lax
from jax.experimental import pallas as pl
from jax.experimental.pallas import tpu as pltpu
from jax.experimental.pallas import tpu_sc as plsc
from jax.experimental.layout import Format, Layout, with_layout_constraint

NUM_EMBEDDINGS = 100000
EMBED = 64
BATCH = 4096
FIELDS = 100

NC = 2
NS = 16
NW = NC * NS

E_PER_W = EMBED // NW
FBLOCKS = 2
FB_FIELDS = FIELDS // FBLOCKS


def _gather_body(idx_hbm, wT_hbm, out_hbm, idx_sh, tbl_v, idx_v, out_v, tsem, isem, osem):
    c = lax.axis_index("c")
    s = lax.axis_index("s")
    wid = s * NC + c

    def wait_idx(b):
        pltpu.make_async_copy(
            idx_sh.at[pl.ds(0, BATCH)], idx_v.at[b], isem.at[b]
        ).wait()

    def wait_out(b):
        pltpu.make_async_copy(
            out_v.at[b], out_hbm.at[0, 0, :, 0, :], osem.at[b]
        ).wait()

    for t in range(E_PER_W):
        e = wid * E_PER_W + t
        er = e // 8
        r = e % 8
        pltpu.async_copy(wT_hbm.at[e], tbl_v, tsem)
        pltpu.make_async_copy(wT_hbm.at[0], tbl_v, tsem).wait()

        for fb in range(FBLOCKS):
            f_base = fb * FB_FIELDS
            plsc.subcore_barrier()

            @pl.when(s == 0)
            def _():
                pltpu.sync_copy(
                    idx_hbm.at[pl.ds(f_base * BATCH, FB_FIELDS * BATCH)], idx_sh
                )

            plsc.subcore_barrier()

            for b in range(2):
                pltpu.async_copy(
                    idx_sh.at[pl.ds(b * BATCH, BATCH)], idx_v.at[b], isem.at[b]
                )

            @pl.loop(0, FB_FIELDS, step=2)
            def _f(f0):
                for b in range(2):
                    fl = f0 + b
                    iv = idx_v.at[b]
                    ov = out_v.at[b]
                    wait_idx(b)

                    @pl.when(f0 + b >= 2)
                    def _():
                        wait_out(b)

                    @plsc.parallel_loop(0, 32, unroll=4)
                    def _i(jj):
                        for k in range(8):
                            ov[jj, pl.ds(k * 16, 16)] = plsc.load_gather(
                                tbl_v, [iv[pl.ds(jj * 128 + k * 16, 16)]]
                            )

                    pltpu.async_copy(
                        ov, out_hbm.at[f_base + fl, er, :, r, :], osem.at[b]
                    )

                    @pl.when(f0 + b + 2 < FB_FIELDS)
                    def _():
                        pltpu.async_copy(
                            idx_sh.at[pl.ds((fl + 2) * BATCH, BATCH)],
                            idx_v.at[b],
                            isem.at[b],
                        )

            for b in range(2):
                wait_out(b)


@jax.jit
def _gather_sc(idxT, wT):
    mesh = plsc.VectorSubcoreMesh(core_axis_name="c", subcore_axis_name="s")
    fn = pl.kernel(
        _gather_body,
        out_type=jax.ShapeDtypeStruct((FIELDS, 8, 32, 8, 128), jnp.float32),
        mesh=mesh,
        scratch_types=[
            pltpu.VMEM_SHARED((FB_FIELDS * BATCH,), jnp.int32),
            pltpu.VMEM((NUM_EMBEDDINGS,), jnp.float32),
            pltpu.VMEM((2, BATCH), jnp.int32),
            pltpu.VMEM((2, 32, 128), jnp.float32),
            pltpu.SemaphoreType.DMA,
            pltpu.SemaphoreType.DMA((2,)),
            pltpu.SemaphoreType.DMA((2,)),
        ],
        compiler_params=pltpu.CompilerParams(
            use_tc_tiling_on_sc=False, needs_layout_passes=False
        ),
    )
    return fn(idxT, wT)


def kernel(x, indices, weight):
    idxT = indices.astype(jnp.int32).T.reshape(-1)
    wT = weight.T
    out = _gather_sc(idxT, wT)
    return out.transpose(2, 4, 0, 1, 3).reshape(BATCH, FIELDS, EMBED)
